# Initial kernel scaffold; baseline (speedup 1.0000x reference)
import jax
import jax.numpy as jnp
from jax.experimental import pallas as pl


def kernel(x, g_ij, idx_i, idx_j, n_atoms, Wf, bf, Wg, Wj, bj, Wi, bi, u, Wres_int, bres_int, Wres_atom, bres_atom, Wres_out, bres_out):
    raise NotImplementedError("write your pallas kernel here")


# speedup vs baseline: 2.3355x; 2.3355x over previous
import functools

import jax
import jax.numpy as jnp
from jax import lax
from jax.experimental import pallas as pl
from jax.experimental.pallas import tpu as pltpu
from jax.experimental.pallas import tpu_sc as plsc

N = 10000
E = 320000
D = 128
NRBF = 32

NC = 2
NS = 16
NW = NC * NS
L = 16

EPW = E // NW
CH = 80
NCHUNK = EPW // CH
ZR = 632
ZL = N - ZR * (NS - 1)

NBLK = 1000
EBLK = 4000


def _silu(t):
    return t * jax.nn.sigmoid(t)


def _mm(a, w):
    return lax.dot_general(a, w, (((1,), (1,)), ((), ())),
                           preferred_element_type=jnp.float32)


def _pre_body(x_ref, wj_ref, bj_ref, wi_ref, bi_ref, u_ref,
              p_ref, vm_ref, xp_ref):
    h = _silu(x_ref[...])
    p_ref[...] = _silu(_mm(h, wj_ref[...]) + bj_ref[...])
    vm_ref[...] = _silu(_mm(h, wi_ref[...]) + bi_ref[...])
    xp_ref[...] = u_ref[...] * h


def _pre(x, Wj, bj, Wi, bi, u):
    grid = N // NBLK
    blk = pl.BlockSpec((NBLK, D), lambda i: (i, 0))
    full = pl.BlockSpec((D, D), lambda i: (0, 0))
    vec = pl.BlockSpec((1, D), lambda i: (0, 0))
    out = jax.ShapeDtypeStruct((N, D), jnp.float32)
    return pl.pallas_call(
        _pre_body,
        grid=(grid,),
        in_specs=[blk, full, vec, full, vec, vec],
        out_specs=[blk, blk, blk],
        out_shape=[out, out, out],
    )(x, Wj, bj.reshape(1, D), Wi, bi.reshape(1, D), u.reshape(1, D))


def _gmat_body(g_ref, wg_ref, o_ref):
    o_ref[...] = _mm(g_ref[...], wg_ref[...])


def _gmat(g_ij, Wg):
    grid = E // EBLK
    return pl.pallas_call(
        _gmat_body,
        grid=(grid,),
        in_specs=[pl.BlockSpec((EBLK, NRBF), lambda i: (i, 0)),
                  pl.BlockSpec((D, NRBF), lambda i: (0, 0))],
        out_specs=pl.BlockSpec((EBLK, D), lambda i: (i, 0)),
        out_shape=jax.ShapeDtypeStruct((E, D), jnp.float32),
    )(g_ij, Wg)


def _edge_body(p_hbm, g_hbm, idxj_hbm, idxi_hbm, zeros_hbm, out_hbm,
               ij_v, ii_v, rows_v, gch_v, acc_sh, sem):
    c = lax.axis_index("c")
    s = lax.axis_index("s")
    wid = c * NS + s

    row0 = pl.multiple_of(s * ZR, 8)

    @pl.when(s < NS - 1)
    def _():
        pltpu.sync_copy(zeros_hbm.at[pl.ds(row0, ZR)],
                        acc_sh.at[pl.ds(row0, ZR)])

    @pl.when(s == NS - 1)
    def _():
        pltpu.sync_copy(zeros_hbm.at[pl.ds(row0, ZL)],
                        acc_sh.at[pl.ds(row0, ZL)])

    plsc.subcore_barrier()

    base = wid * EPW

    def chunk(k, carry):
        off = pl.multiple_of(base + k * CH, 8)
        pltpu.sync_copy(idxj_hbm.at[pl.ds(off, CH)], ij_v)
        pltpu.sync_copy(idxi_hbm.at[pl.ds(off, CH)], ii_v)
        pltpu.async_copy(p_hbm.at[ij_v], rows_v, sem).wait()
        pltpu.sync_copy(g_hbm.at[pl.ds(off, CH)], gch_v)

        def row(r, carry2):
            for j in range(D // L):
                sl = pl.ds(j * L, L)
                rows_v[r, sl] = rows_v[r, sl] * gch_v[r, sl]
            return carry2

        lax.fori_loop(0, CH, row, 0)
        pltpu.sync_copy(rows_v, acc_sh.at[ii_v], add=True)
        return carry

    lax.fori_loop(0, NCHUNK, chunk, 0)
    plsc.subcore_barrier()

    @pl.when(s < NS - 1)
    def _():
        pltpu.sync_copy(acc_sh.at[pl.ds(row0, ZR)],
                        out_hbm.at[c, pl.ds(row0, ZR)])

    @pl.when(s == NS - 1)
    def _():
        pltpu.sync_copy(acc_sh.at[pl.ds(row0, ZL)],
                        out_hbm.at[c, pl.ds(row0, ZL)])


@functools.cache
def _edge_kernel():
    return pl.kernel(
        _edge_body,
        mesh=plsc.VectorSubcoreMesh(core_axis_name="c", subcore_axis_name="s",
                                    num_cores=NC, num_subcores=NS),
        out_type=jax.ShapeDtypeStruct((NC, N, D), jnp.float32),
        scratch_types=[
            pltpu.VMEM((CH,), jnp.int32),
            pltpu.VMEM((CH,), jnp.int32),
            pltpu.VMEM((CH, D), jnp.float32),
            pltpu.VMEM((CH, D), jnp.float32),
            pltpu.VMEM_SHARED((N, D), jnp.float32),
            pltpu.SemaphoreType.DMA,
        ],
    )


def _edge(p, g, idxj, idxi, zeros):
    return _edge_kernel()(p, g, idxj, idxi, zeros)


def _res_block(h, w, b):
    t = _silu(h)
    t = _mm(t, w) + b
    t = _silu(t)
    t = _mm(t, w) + b
    return t + h


def _post_body(v0_ref, v1_ref, vm_ref, xp_ref, wf_ref, bf_ref,
               wri_ref, bri_ref, wra_ref, bra_ref, wro_ref, bro_ref,
               o_ref, h_ref):
    v = v0_ref[...] + v1_ref[...] + vm_ref[...]
    for i in range(3):
        v = _res_block(v, wri_ref[i], bri_ref[i])
    v = _silu(v)
    h = xp_ref[...] + _mm(v, wf_ref[...]) + bf_ref[...]
    h = _res_block(h, wra_ref[0], bra_ref[0])
    o = _res_block(h, wro_ref[0], bro_ref[0])
    o_ref[...] = _silu(o)
    h_ref[...] = h


def _post(v0, v1, vm, xp, Wf, bf, Wres_int, bres_int,
          Wres_atom, bres_atom, Wres_out, bres_out):
    grid = N // NBLK
    blk = pl.BlockSpec((NBLK, D), lambda i: (i, 0))
    full = pl.BlockSpec((D, D), lambda i: (0, 0))
    vec = pl.BlockSpec((1, D), lambda i: (0, 0))
    w3 = pl.BlockSpec((3, D, D), lambda i: (0, 0, 0))
    b3 = pl.BlockSpec((3, 1, D), lambda i: (0, 0, 0))
    w1 = pl.BlockSpec((1, D, D), lambda i: (0, 0, 0))
    b1 = pl.BlockSpec((1, 1, D), lambda i: (0, 0, 0))
    out = jax.ShapeDtypeStruct((N, D), jnp.float32)
    return pl.pallas_call(
        _post_body,
        grid=(grid,),
        in_specs=[blk, blk, blk, blk, full, vec, w3, b3, w1, b1, w1, b1],
        out_specs=[blk, blk],
        out_shape=[out, out],
    )(v0, v1, vm, xp, Wf, bf.reshape(1, D),
      Wres_int, bres_int.reshape(3, 1, D),
      Wres_atom, bres_atom.reshape(1, 1, D),
      Wres_out, bres_out.reshape(1, 1, D))


def kernel(x, g_ij, idx_i, idx_j, n_atoms, Wf, bf, Wg, Wj, bj, Wi, bi, u,
           Wres_int, bres_int, Wres_atom, bres_atom, Wres_out, bres_out):
    del n_atoms
    P, vm, xp = _pre(x, Wj, bj, Wi, bi, u)
    G = _gmat(g_ij, Wg)
    zeros = jnp.zeros((N, D), dtype=jnp.float32)
    part = _edge(P, G, idx_j.astype(jnp.int32), idx_i.astype(jnp.int32), zeros)
    o, h = _post(part[0], part[1], vm, xp, Wf, bf, Wres_int, bres_int,
                 Wres_atom, bres_atom, Wres_out, bres_out)
    return (o, h)
